# baseline (speedup 1.0000x reference)
import jax
import jax.numpy as jnp
from jax import lax
from jax.experimental import pallas as pl
from jax.experimental.pallas import tpu as pltpu
from jax.experimental.pallas import tpu_sc as plsc

N = 10000
E = 320000
D = 128
S = 2048
DG = 8
DE = D + DG

NC = 2
NS = 16
NW = NC * NS
EW = E // NW
K = 100
CPS = 10
NSUPER = EW // (K * CPS)
ROWS_T = N // NS
SW = S // NW
LANES = 16

ROW_OFFS = tuple(range(0, D, LANES)) + (DE - LANES,)
IDX_OFFS = tuple(range(0, K - LANES + 1, LANES)) + (K - LANES,)


def _encode_body(src_h, dst_h, typ_h, nodep_h, relnp_h,
                 agg0_h, agg1_h,
                 agg_sh, rel_sh, sidx, didx, tidx,
                 br0, br1, br2,
                 semb0, semb1, semb2, semc0, semc1, semc2,
                 semd0, semd1, semd2):
    c = lax.axis_index("c")
    s = lax.axis_index("s")
    wid = s * NC + c
    br = [br0, br1, br2]
    semb = [semb0, semb1, semb2]
    semc = [semc0, semc1, semc2]
    semd = [semd0, semd1, semd2]
    zero = jnp.zeros((LANES,), jnp.float32)
    zeroi = jnp.zeros((LANES,), jnp.int32)

    def zb_body(j, carry):
        for off in ROW_OFFS:
            br0[j, pl.ds(off, LANES)] = zero
        return carry

    lax.fori_loop(0, K, zb_body, None)

    def didx_body(j, carry):
        for off in IDX_OFFS:
            didx[j, pl.ds(off, LANES)] = zeroi
        return carry

    lax.fori_loop(0, CPS, didx_body, None)

    for q in range(ROWS_T // K):
        pltpu.sync_copy(br0, agg_sh.at[pl.ds(s * ROWS_T + q * K, K)])
    pltpu.sync_copy(br0.at[pl.ds(0, ROWS_T % K)],
                    agg_sh.at[pl.ds(s * ROWS_T + (ROWS_T // K) * K,
                                    ROWS_T % K)])

    @pl.when(s == 0)
    def _():
        pltpu.sync_copy(relnp_h, rel_sh)
    plsc.subcore_barrier()

    def block(g, carry):
        rbase = wid * (EW // K) + g * CPS
        pltpu.sync_copy(src_h.at[pl.ds(rbase, CPS)], sidx)
        pltpu.sync_copy(typ_h.at[pl.ds(rbase, CPS)], tidx)
        pltpu.sync_copy(dst_h.at[pl.ds(rbase, CPS)], didx)
        cpn = [None, None, None]
        cpc = [None, None, None]
        cpn[0] = pltpu.async_copy(nodep_h.at[sidx.at[0]], br[0], semb[0])
        for q in range(CPS):
            p = q % 3
            if q >= 1:
                pl_ = (q - 1) % 3
                cpc[pl_].wait()
                pltpu.async_copy(br[pl_], agg_sh.at[didx.at[q - 1]],
                                 semd[pl_], add=True)
            if q >= 2:
                pp = (q - 2) % 3
                pltpu.make_async_copy(
                    br0, agg_sh.at[didx.at[0]], semd[pp]).wait()
            if q + 1 < CPS:
                pn = (q + 1) % 3
                cpn[pn] = pltpu.async_copy(nodep_h.at[sidx.at[q + 1]],
                                           br[pn], semb[pn])
            cpn[p].wait()
            cpc[p] = pltpu.async_copy(rel_sh.at[tidx.at[q]], br[p],
                                      semc[p], add=True)
        pl_ = (CPS - 1) % 3
        cpc[pl_].wait()
        pltpu.async_copy(br[pl_], agg_sh.at[didx.at[CPS - 1]],
                         semd[pl_], add=True)
        for pp in ((CPS - 2) % 3, (CPS - 1) % 3):
            pltpu.make_async_copy(br0, agg_sh.at[didx.at[0]], semd[pp]).wait()
        return carry

    lax.fori_loop(0, NSUPER, block, None)
    plsc.subcore_barrier()

    @pl.when(c == 0)
    def _():
        pltpu.sync_copy(agg_sh.at[pl.ds(s * ROWS_T, ROWS_T)],
                        agg0_h.at[pl.ds(s * ROWS_T, ROWS_T)])

    @pl.when(c == 1)
    def _():
        pltpu.sync_copy(agg_sh.at[pl.ds(s * ROWS_T, ROWS_T)],
                        agg1_h.at[pl.ds(s * ROWS_T, ROWS_T)])


def _extract_body(sub_h, a0_h, a1_h, node_h, x_h,
                  idx, g0, g1, gn, xb, sem):
    c = lax.axis_index("c")
    s = lax.axis_index("s")
    wid = s * NC + c
    base = wid * SW
    pltpu.sync_copy(sub_h.at[pl.ds(base, SW)], idx)
    cps = [pltpu.async_copy(a0_h.at[idx], g0, sem),
           pltpu.async_copy(a1_h.at[idx], g1, sem),
           pltpu.async_copy(node_h.at[idx], gn, sem)]
    for cp in cps:
        cp.wait()

    def row(j, carry):
        dpos = jnp.full((LANES,), D, jnp.int32)
        jv = jnp.full((LANES,), j, jnp.int32)
        deg = (plsc.load_gather(g0, [jv, dpos]) +
               plsc.load_gather(g1, [jv, dpos]))
        rcp = 1.0 / jnp.maximum(deg, 1.0)
        for i in range(D // LANES):
            sl = pl.ds(i * LANES, LANES)
            xb[j, sl] = (g0[j, sl] + g1[j, sl]) * rcp + gn[j, sl]
        return carry

    lax.fori_loop(0, SW, row, None)
    pltpu.sync_copy(xb, x_h.at[pl.ds(base, SW)])


def _matmul_body(x_ref, w_ref, o_ref):
    o_ref[...] = jnp.tanh(
        jnp.dot(x_ref[...], w_ref[...], preferred_element_type=jnp.float32))


@jax.jit
def kernel(edge_index, edge_type, subgraph_nodes, node_emb, rel_emb, W):
    src = edge_index[0].reshape(E // K, K)
    dst = edge_index[1].reshape(E // K, K)
    etype = edge_type.reshape(E // K, K)
    node_pad = jnp.concatenate(
        [node_emb, jnp.ones((N, DG), jnp.float32)], axis=1)
    reln_pad = jnp.concatenate(
        [-rel_emb, jnp.zeros((rel_emb.shape[0], DG), jnp.float32)], axis=1)

    mesh = plsc.VectorSubcoreMesh(core_axis_name="c", subcore_axis_name="s")
    encode = pl.kernel(
        _encode_body,
        out_type=[jax.ShapeDtypeStruct((N, DE), jnp.float32),
                  jax.ShapeDtypeStruct((N, DE), jnp.float32)],
        mesh=mesh,
        scratch_types=[
            pltpu.VMEM_SHARED((N, DE), jnp.float32),
            pltpu.VMEM_SHARED((200, DE), jnp.float32),
            pltpu.VMEM((CPS, K), jnp.int32),
            pltpu.VMEM((CPS, K), jnp.int32),
            pltpu.VMEM((CPS, K), jnp.int32),
            pltpu.VMEM((K, DE), jnp.float32),
            pltpu.VMEM((K, DE), jnp.float32),
            pltpu.VMEM((K, DE), jnp.float32),
        ] + [pltpu.SemaphoreType.DMA] * 9,
        compiler_params=pltpu.CompilerParams(use_tc_tiling_on_sc=False),
    )
    agg0, agg1 = encode(src, dst, etype, node_pad, reln_pad)

    extract = pl.kernel(
        _extract_body,
        out_type=jax.ShapeDtypeStruct((S, D), jnp.float32),
        mesh=plsc.VectorSubcoreMesh(core_axis_name="c", subcore_axis_name="s"),
        scratch_types=[
            pltpu.VMEM((SW,), jnp.int32),
            pltpu.VMEM((SW, DE), jnp.float32),
            pltpu.VMEM((SW, DE), jnp.float32),
            pltpu.VMEM((SW, D), jnp.float32),
            pltpu.VMEM((SW, D), jnp.float32),
            pltpu.SemaphoreType.DMA,
        ],
        compiler_params=pltpu.CompilerParams(use_tc_tiling_on_sc=False,
                                             needs_layout_passes=False),
    )
    x = extract(subgraph_nodes, agg0, agg1, node_emb)

    return pl.pallas_call(
        _matmul_body,
        out_shape=jax.ShapeDtypeStruct((S, D), jnp.float32),
    )(x, W)


# speedup vs baseline: 1.2131x; 1.2131x over previous
import jax
import jax.numpy as jnp
from jax import lax
from jax.experimental import pallas as pl
from jax.experimental.pallas import tpu as pltpu
from jax.experimental.pallas import tpu_sc as plsc

N = 10000
E = 320000
D = 128
S = 2048
DG = 8
DE = D + DG

NC = 2
NS = 16
NW = NC * NS
EW = E // NW
K = 80
CPS = 25
NSUPER = EW // (K * CPS)
ROWS_T = N // NS
SW = S // NW
LANES = 16

ROW_OFFS = tuple(range(0, D, LANES)) + (DE - LANES,)


def _encode_body(src_h, dst_h, typ_h, nodep_h, relnp_h,
                 agg0_h, agg1_h,
                 agg_sh, rel_sh, sidx, didx, tidx,
                 br0, br1, br2,
                 semb0, semb1, semb2, semc0, semc1, semc2,
                 semd0, semd1, semd2):
    c = lax.axis_index("c")
    s = lax.axis_index("s")
    wid = s * NC + c
    br = [br0, br1, br2]
    semb = [semb0, semb1, semb2]
    semc = [semc0, semc1, semc2]
    semd = [semd0, semd1, semd2]
    zero = jnp.zeros((LANES,), jnp.float32)
    zeroi = jnp.zeros((LANES,), jnp.int32)

    def zb_body(j, carry):
        for off in ROW_OFFS:
            br0[j, pl.ds(off, LANES)] = zero
        return carry

    lax.fori_loop(0, K, zb_body, None)

    def didx_body(j, carry):
        for i in range(K // LANES):
            didx[j, pl.ds(i * LANES, LANES)] = zeroi
        return carry

    lax.fori_loop(0, CPS, didx_body, None)

    for q in range(7):
        pltpu.sync_copy(br0, agg_sh.at[pl.ds(s * ROWS_T + q * K, K)])
    pltpu.sync_copy(br0.at[pl.ds(0, 65)],
                    agg_sh.at[pl.ds(s * ROWS_T + 7 * K, 65)])

    @pl.when(s == 0)
    def _():
        pltpu.sync_copy(relnp_h, rel_sh)
    plsc.subcore_barrier()

    def block(g, carry):
        rbase = wid * (EW // K) + g * CPS
        pltpu.sync_copy(src_h.at[pl.ds(rbase, CPS)], sidx)
        pltpu.sync_copy(typ_h.at[pl.ds(rbase, CPS)], tidx)
        pltpu.sync_copy(dst_h.at[pl.ds(rbase, CPS)], didx)
        cpn = [None, None, None]
        cpc = [None, None, None]
        cpn[0] = pltpu.async_copy(nodep_h.at[sidx.at[0]], br[0], semb[0])
        for q in range(CPS):
            p = q % 3
            if q >= 1:
                pl_ = (q - 1) % 3
                cpc[pl_].wait()
                pltpu.async_copy(br[pl_], agg_sh.at[didx.at[q - 1]],
                                 semd[pl_], add=True)
            if q >= 2:
                pp = (q - 2) % 3
                pltpu.make_async_copy(
                    br0, agg_sh.at[didx.at[0]], semd[pp]).wait()
            if q + 1 < CPS:
                pn = (q + 1) % 3
                cpn[pn] = pltpu.async_copy(nodep_h.at[sidx.at[q + 1]],
                                           br[pn], semb[pn])
            cpn[p].wait()
            cpc[p] = pltpu.async_copy(rel_sh.at[tidx.at[q]], br[p],
                                      semc[p], add=True)
        pl_ = (CPS - 1) % 3
        cpc[pl_].wait()
        pltpu.async_copy(br[pl_], agg_sh.at[didx.at[CPS - 1]],
                         semd[pl_], add=True)
        for pp in ((CPS - 2) % 3, (CPS - 1) % 3):
            pltpu.make_async_copy(br0, agg_sh.at[didx.at[0]], semd[pp]).wait()
        return carry

    lax.fori_loop(0, NSUPER, block, None)
    plsc.subcore_barrier()

    @pl.when(c == 0)
    def _():
        pltpu.sync_copy(agg_sh.at[pl.ds(s * ROWS_T, ROWS_T)],
                        agg0_h.at[pl.ds(s * ROWS_T, ROWS_T)])

    @pl.when(c == 1)
    def _():
        pltpu.sync_copy(agg_sh.at[pl.ds(s * ROWS_T, ROWS_T)],
                        agg1_h.at[pl.ds(s * ROWS_T, ROWS_T)])


def _extract_body(sub_h, a0_h, a1_h, node_h, x_h,
                  idx, g0, g1, gn, xb, sem):
    c = lax.axis_index("c")
    s = lax.axis_index("s")
    wid = s * NC + c
    base = wid * SW
    pltpu.sync_copy(sub_h.at[pl.ds(base, SW)], idx)
    cps = [pltpu.async_copy(a0_h.at[idx], g0, sem),
           pltpu.async_copy(a1_h.at[idx], g1, sem),
           pltpu.async_copy(node_h.at[idx], gn, sem)]
    for cp in cps:
        cp.wait()

    def row(j, carry):
        dpos = jnp.full((LANES,), D, jnp.int32)
        jv = jnp.full((LANES,), j, jnp.int32)
        deg = (plsc.load_gather(g0, [jv, dpos]) +
               plsc.load_gather(g1, [jv, dpos]))
        rcp = 1.0 / jnp.maximum(deg, 1.0)
        for i in range(D // LANES):
            sl = pl.ds(i * LANES, LANES)
            xb[j, sl] = (g0[j, sl] + g1[j, sl]) * rcp + gn[j, sl]
        return carry

    lax.fori_loop(0, SW, row, None)
    pltpu.sync_copy(xb, x_h.at[pl.ds(base, SW)])


def _matmul_body(x_ref, w_ref, o_ref):
    o_ref[...] = jnp.tanh(
        jnp.dot(x_ref[...], w_ref[...], preferred_element_type=jnp.float32))


@jax.jit
def kernel(edge_index, edge_type, subgraph_nodes, node_emb, rel_emb, W):
    src = edge_index[0].reshape(E // K, K)
    dst = edge_index[1].reshape(E // K, K)
    etype = edge_type.reshape(E // K, K)
    node_pad = jnp.concatenate(
        [node_emb, jnp.ones((N, DG), jnp.float32)], axis=1)
    reln_pad = jnp.concatenate(
        [-rel_emb, jnp.zeros((rel_emb.shape[0], DG), jnp.float32)], axis=1)

    mesh = plsc.VectorSubcoreMesh(core_axis_name="c", subcore_axis_name="s")
    encode = pl.kernel(
        _encode_body,
        out_type=[jax.ShapeDtypeStruct((N, DE), jnp.float32),
                  jax.ShapeDtypeStruct((N, DE), jnp.float32)],
        mesh=mesh,
        scratch_types=[
            pltpu.VMEM_SHARED((N, DE), jnp.float32),
            pltpu.VMEM_SHARED((200, DE), jnp.float32),
            pltpu.VMEM((CPS, K), jnp.int32),
            pltpu.VMEM((CPS, K), jnp.int32),
            pltpu.VMEM((CPS, K), jnp.int32),
            pltpu.VMEM((K, DE), jnp.float32),
            pltpu.VMEM((K, DE), jnp.float32),
            pltpu.VMEM((K, DE), jnp.float32),
        ] + [pltpu.SemaphoreType.DMA] * 9,
        compiler_params=pltpu.CompilerParams(use_tc_tiling_on_sc=False),
    )
    agg0, agg1 = encode(src, dst, etype, node_pad, reln_pad)

    extract = pl.kernel(
        _extract_body,
        out_type=jax.ShapeDtypeStruct((S, D), jnp.float32),
        mesh=plsc.VectorSubcoreMesh(core_axis_name="c", subcore_axis_name="s"),
        scratch_types=[
            pltpu.VMEM((SW,), jnp.int32),
            pltpu.VMEM((SW, DE), jnp.float32),
            pltpu.VMEM((SW, DE), jnp.float32),
            pltpu.VMEM((SW, D), jnp.float32),
            pltpu.VMEM((SW, D), jnp.float32),
            pltpu.SemaphoreType.DMA,
        ],
        compiler_params=pltpu.CompilerParams(use_tc_tiling_on_sc=False,
                                             needs_layout_passes=False),
    )
    x = extract(subgraph_nodes, agg0, agg1, node_emb)

    return pl.pallas_call(
        _matmul_body,
        out_shape=jax.ShapeDtypeStruct((S, D), jnp.float32),
    )(x, W)
